# baseline (speedup 1.0000x reference)
import jax
import jax.numpy as jnp
from jax import lax
from jax.experimental import pallas as pl
from jax.experimental.pallas import tpu as pltpu
from jax.experimental.pallas import tpu_sc as plsc

B = 1024
T = 200
D = 128
LANES = 16
NUM_CORES = 2
NUM_SUBCORES = 16
NUM_WORKERS = NUM_CORES * NUM_SUBCORES
TOK_PER_WORKER = B * T // NUM_WORKERS
CHUNK = 128
NCHUNK = TOK_PER_WORKER // CHUNK
VREGS_PER_ROW = D // LANES
NBUF = 5
MAXP0 = max((CHUNK * c) % T for c in range(NCHUNK))
POS_EXT = MAXP0 + CHUNK


def _body(x_hbm, tok_hbm, pos_hbm, out_hbm, idx_v, pos_v, bufs, gsems,
          osems):
    wid = lax.axis_index("s") * NUM_CORES + lax.axis_index("c")
    chunk0 = wid * NCHUNK

    pltpu.sync_copy(x_hbm.at[wid], idx_v)
    pltpu.sync_copy(pos_hbm.at[pl.ds(0, T)], pos_v.at[pl.ds(0, T)])
    pltpu.sync_copy(pos_hbm.at[pl.ds(0, POS_EXT - T)],
                    pos_v.at[pl.ds(T, POS_EXT - T)])

    def fire_gather(c):
        pltpu.async_copy(tok_hbm.at[idx_v.at[c]], bufs[c % NBUF],
                         gsems[c % NBUF])

    def drain_gather(c):
        pltpu.make_async_copy(tok_hbm.at[idx_v.at[c]], bufs[c % NBUF],
                              gsems[c % NBUF]).wait()

    def fire_out(c):
        pltpu.async_copy(bufs[c % NBUF], out_hbm.at[chunk0 + c],
                         osems[c % NBUF])

    def wait_out(c):
        pltpu.make_async_copy(bufs[c % NBUF], out_hbm.at[chunk0 + c],
                              osems[c % NBUF]).wait()

    for c in range(NBUF - 1):
        fire_gather(c)
    for c in range(NCHUNK):
        buf = bufs[c % NBUF]
        drain_gather(c)

        p0 = (CHUNK * c) % T

        def add_row(j, _):
            for v in range(VREGS_PER_ROW):
                sl = pl.ds(v * LANES, LANES)
                plsc.addupdate(buf.at[j, sl], pos_v[p0 + j, sl])
            return 0

        lax.fori_loop(0, CHUNK, add_row, 0)

        fire_out(c)
        if c + NBUF - 1 < NCHUNK:
            if c >= 1:
                wait_out(c - 1)
            fire_gather(c + NBUF - 1)
    for c in range(NCHUNK - NBUF, NCHUNK):
        wait_out(c)


@jax.jit
def kernel(x, token_table, pos_table):
    mesh = plsc.VectorSubcoreMesh(
        core_axis_name="c", subcore_axis_name="s",
        num_cores=NUM_CORES, num_subcores=NUM_SUBCORES)

    def body(x_hbm, tok_hbm, pos_hbm, out_hbm, idx_v, pos_v, *rest):
        return _body(x_hbm, tok_hbm, pos_hbm, out_hbm, idx_v, pos_v,
                     rest[:NBUF], rest[NBUF:2 * NBUF], rest[2 * NBUF:])

    run = pl.kernel(
        body,
        out_type=jax.ShapeDtypeStruct((B * T // CHUNK, CHUNK, D),
                                      jnp.float32),
        mesh=mesh,
        scratch_types=(
            [pltpu.VMEM((NCHUNK, CHUNK), jnp.int32),
             pltpu.VMEM((POS_EXT, D), jnp.float32)]
            + [pltpu.VMEM((CHUNK, D), jnp.float32)] * NBUF
            + [pltpu.SemaphoreType.DMA] * (2 * NBUF)
        ),
    )
    out = run(x.reshape(NUM_WORKERS, NCHUNK, CHUNK), token_table, pos_table)
    return out.reshape(B, T, D)


# speedup vs baseline: 1.0344x; 1.0344x over previous
import jax
import jax.numpy as jnp
from jax import lax
from jax.experimental import pallas as pl
from jax.experimental.pallas import tpu as pltpu
from jax.experimental.pallas import tpu_sc as plsc

B = 1024
T = 200
D = 128
LANES = 16
NUM_CORES = 2
NUM_SUBCORES = 16
NUM_WORKERS = NUM_CORES * NUM_SUBCORES
ROWS_PER_WORKER = B // NUM_WORKERS
SPLIT = 128
REST = T - SPLIT
VREGS_PER_ROW = D // LANES
NBUF = 3


def _body(x_hbm, tok_hbm, pos_hbm, out_hbm, idx_v, pos_v, buf0, buf1, buf2,
          g0, g1, g2, o0, o1, o2):
    wid = lax.axis_index("s") * NUM_CORES + lax.axis_index("c")
    row0 = wid * ROWS_PER_WORKER

    pltpu.sync_copy(x_hbm.at[pl.ds(row0, ROWS_PER_WORKER)], idx_v)
    pltpu.sync_copy(pos_hbm.at[pl.ds(0, T)], pos_v)

    bufs = (buf0, buf1, buf2)
    gsems = (g0, g1, g2)
    osems = (o0, o1, o2)

    def fire_gather(r):
        buf, sem = bufs[r % NBUF], gsems[r % NBUF]
        pltpu.async_copy(tok_hbm.at[idx_v.at[r, pl.ds(0, SPLIT)]],
                         buf.at[pl.ds(0, SPLIT)], sem)
        pltpu.async_copy(tok_hbm.at[idx_v.at[r, pl.ds(SPLIT, REST)]],
                         buf.at[pl.ds(SPLIT, REST)], sem)

    def drain_gather(r):
        buf, sem = bufs[r % NBUF], gsems[r % NBUF]
        pltpu.make_async_copy(tok_hbm.at[idx_v.at[r, pl.ds(0, SPLIT)]],
                              buf.at[pl.ds(0, SPLIT)], sem).wait()
        pltpu.make_async_copy(tok_hbm.at[idx_v.at[r, pl.ds(SPLIT, REST)]],
                              buf.at[pl.ds(SPLIT, REST)], sem).wait()

    def fire_out_part(r, lo, n):
        buf, sem = bufs[r % NBUF], osems[r % NBUF]
        pltpu.async_copy(buf.at[pl.ds(lo, n)],
                         out_hbm.at[row0 + r, pl.ds(lo, n)], sem)

    def wait_out(r):
        buf, sem = bufs[r % NBUF], osems[r % NBUF]
        pltpu.make_async_copy(buf.at[pl.ds(0, SPLIT)],
                              out_hbm.at[row0 + r, pl.ds(0, SPLIT)],
                              sem).wait()
        pltpu.make_async_copy(buf.at[pl.ds(SPLIT, REST)],
                              out_hbm.at[row0 + r, pl.ds(SPLIT, REST)],
                              sem).wait()

    def add_rows(buf, lo, hi):
        def add_row(j, _):
            for v in range(VREGS_PER_ROW):
                sl = pl.ds(v * LANES, LANES)
                plsc.addupdate(buf.at[j, sl], pos_v[j, sl])
            return 0
        lax.fori_loop(lo, hi, add_row, 0)

    fire_gather(0)
    fire_gather(1)
    for r in range(ROWS_PER_WORKER):
        buf = bufs[r % NBUF]
        drain_gather(r)
        if r + 2 < ROWS_PER_WORKER:
            if r >= 1:
                wait_out(r - 1)
            fire_gather(r + 2)
        add_rows(buf, 0, SPLIT)
        fire_out_part(r, 0, SPLIT)
        add_rows(buf, SPLIT, T)
        fire_out_part(r, SPLIT, REST)
    for r in range(ROWS_PER_WORKER - NBUF, ROWS_PER_WORKER):
        wait_out(r)


@jax.jit
def kernel(x, token_table, pos_table):
    mesh = plsc.VectorSubcoreMesh(
        core_axis_name="c", subcore_axis_name="s",
        num_cores=NUM_CORES, num_subcores=NUM_SUBCORES)
    run = pl.kernel(
        _body,
        out_type=jax.ShapeDtypeStruct((B, T, D), jnp.float32),
        mesh=mesh,
        scratch_types=[
            pltpu.VMEM((ROWS_PER_WORKER, T), jnp.int32),
            pltpu.VMEM((T, D), jnp.float32),
            pltpu.VMEM((T, D), jnp.float32),
            pltpu.VMEM((T, D), jnp.float32),
            pltpu.VMEM((T, D), jnp.float32),
            pltpu.SemaphoreType.DMA,
            pltpu.SemaphoreType.DMA,
            pltpu.SemaphoreType.DMA,
            pltpu.SemaphoreType.DMA,
            pltpu.SemaphoreType.DMA,
            pltpu.SemaphoreType.DMA,
        ],
    )
    return run(x, token_table, pos_table)
